# baseline (speedup 1.0000x reference)
import functools

import jax
import jax.numpy as jnp
from jax import lax
from jax.experimental import pallas as pl
from jax.experimental.pallas import tpu as pltpu
from jax.experimental.pallas import tpu_sc as plsc

_NW = 32
_BC = 512
_L = 16


def _emb_body(units_per_w, ids_t, table_hbm, out_t,
              jsbuf, idx0, idx1, rows0, rows1, blk0, blk1,
              g0, g1, o0, o1):
    s_total, b_total = ids_t.shape
    d = table_hbm.shape[1]
    nbc = b_total // _BC
    wid = lax.axis_index("s") * 2 + lax.axis_index("c")
    u0 = wid * units_per_w

    for l in range(_BC // _L):
        jsbuf[pl.ds(l * _L, _L)] = lax.iota(jnp.int32, _L) + l * _L

    def unit_sb(u):
        return u // nbc, (u % nbc) * _BC

    def i_copy(u, idx):
        s, b = unit_sb(u)
        return pltpu.make_async_copy(ids_t.at[s, pl.ds(b, _BC)], idx, None)

    def g_copy(u, idx, rows, sem):
        return pltpu.make_async_copy(table_hbm.at[idx], rows, sem)

    def s_copy(u, blk, sem):
        s, b = unit_sb(u)
        return pltpu.make_async_copy(blk, out_t.at[s, :, pl.ds(b, _BC)], sem)

    def extract(rows, blk):
        def fbody(f, fv):
            for l in range(_BC // _L):
                blk[f, pl.ds(l * _L, _L)] = plsc.load_gather(
                    rows, [jsbuf[pl.ds(l * _L, _L)], fv])
            return fv + 1

        lax.fori_loop(0, d, fbody, jnp.zeros((_L,), jnp.int32), unroll=False)

    def stage(u, idx, rows, gsem):
        pltpu.sync_copy(ids_t.at[unit_sb(u)[0], pl.ds(unit_sb(u)[1], _BC)],
                        idx)
        g_copy(u, idx, rows, gsem).start()

    stage(u0, idx0, rows0, g0)
    stage(u0 + 1, idx1, rows1, g1)

    def body(i, carry):
        u = u0 + 2 * i

        g_copy(u, idx0, rows0, g0).wait()

        @pl.when(i > 0)
        def _():
            s_copy(u, blk0, o0).wait()

        extract(rows0, blk0)
        s_copy(u, blk0, o0).start()

        @pl.when(2 * i + 2 < units_per_w)
        def _():
            stage(u + 2, idx0, rows0, g0)

        g_copy(u + 1, idx1, rows1, g1).wait()

        @pl.when(i > 0)
        def _():
            s_copy(u + 1, blk1, o1).wait()

        extract(rows1, blk1)
        s_copy(u + 1, blk1, o1).start()

        @pl.when(2 * i + 3 < units_per_w)
        def _():
            stage(u + 3, idx1, rows1, g1)

        return carry

    lax.fori_loop(0, units_per_w // 2, body, 0, unroll=False)
    s_copy(u0, blk0, o0).wait()
    s_copy(u0, blk1, o1).wait()


def kernel(token_ids, table):
    b, s = token_ids.shape
    _, d = table.shape
    nbc = b // _BC
    units = s * nbc
    assert units % (2 * _NW) == 0 and b % _BC == 0

    ids_t = token_ids.T
    mesh = plsc.VectorSubcoreMesh(core_axis_name="c", subcore_axis_name="s")
    k = pl.kernel(
        functools.partial(_emb_body, units // _NW),
        out_type=jax.ShapeDtypeStruct((s, d, b), jnp.float32),
        mesh=mesh,
        scratch_types=[
            pltpu.VMEM((_BC,), jnp.int32),
            pltpu.VMEM((_BC,), jnp.int32),
            pltpu.VMEM((_BC,), jnp.int32),
            pltpu.VMEM((_BC, d), jnp.float32),
            pltpu.VMEM((_BC, d), jnp.float32),
            pltpu.VMEM((d, _BC), jnp.float32),
            pltpu.VMEM((d, _BC), jnp.float32),
            pltpu.SemaphoreType.DMA,
            pltpu.SemaphoreType.DMA,
            pltpu.SemaphoreType.DMA,
            pltpu.SemaphoreType.DMA,
        ],
        compiler_params=pltpu.CompilerParams(
            use_tc_tiling_on_sc=False, needs_layout_passes=False),
    )
    out_t = k(ids_t, table)
    return jnp.transpose(out_t, (2, 0, 1))
